# baseline (speedup 1.0000x reference)
import functools

import numpy as np
import jax
import jax.numpy as jnp
from jax import lax
from jax.experimental import pallas as pl
from jax.experimental.pallas import tpu as pltpu
from jax.experimental.pallas import tpu_sc as plsc

NUM_SLOTS_PER_BATCH = 64
_FEAT = 256
_FILL = -1000000.0

_ROT_A = (13, 15, 26, 6)
_ROT_B = (17, 29, 16, 24)


def _np_threefry2x32(k0, k1, x0, x1):
    k0 = np.uint32(k0); k1 = np.uint32(k1)
    k2 = np.uint32(k0 ^ k1 ^ np.uint32(0x1BD11BDA))
    ks = [k0, k1, k2]
    rots = [_ROT_A, _ROT_B]
    with np.errstate(over="ignore"):
        x0 = np.uint32(x0 + k0); x1 = np.uint32(x1 + k1)
        for i in range(5):
            for r in rots[i % 2]:
                x0 = np.uint32(x0 + x1)
                x1 = np.uint32((x1 << np.uint32(r)) | (x1 >> np.uint32(32 - r)))
                x1 = np.uint32(x0 ^ x1)
            x0 = np.uint32(x0 + ks[(i + 1) % 3])
            x1 = np.uint32(x1 + ks[(i + 2) % 3] + np.uint32(i + 1))
    return x0, x1


_KS0_NP, _KS1_NP = _np_threefry2x32(0, 42, np.uint32(0), np.uint32(2))
_KS0 = np.uint32(_KS0_NP)
_KS1 = np.uint32(_KS1_NP)
_KS2 = np.uint32(_KS0 ^ _KS1 ^ np.uint32(0x1BD11BDA))

_ERFINV_SMALL = (2.81022636e-08, 3.43273939e-07, -3.5233877e-06,
                 -4.39150654e-06, 0.00021858087, -0.00125372503,
                 -0.00417768164, 0.246640727, 1.50140941)
_ERFINV_BIG = (-0.000200214257, 0.000100950558, 0.00134934322,
               -0.00367342844, 0.00573950773, -0.0076224613,
               0.00943887047, 1.00167406, 2.83297682)

_LO = np.float32(np.nextafter(np.float32(-1.0), np.float32(0.0)))
_SQRT2 = np.float32(np.sqrt(2.0))


def _tf_rounds(x0, x1):
    ks = (_KS0, _KS1, _KS2)
    rots = (_ROT_A, _ROT_B)
    x0 = x0 + _KS0
    x1 = x1 + _KS1
    for i in range(5):
        for r in rots[i % 2]:
            x0 = x0 + x1
            x1 = lax.shift_left(x1, np.uint32(r)) | lax.shift_right_logical(
                x1, np.uint32(32 - r))
            x1 = x0 ^ x1
        x0 = x0 + ks[(i + 1) % 3]
        x1 = x1 + ks[(i + 2) % 3] + np.uint32(i + 1)
    return x0, x1


def _horner(coeffs, t):
    p = jnp.full(t.shape, np.float32(coeffs[0]), dtype=jnp.float32)
    for c in coeffs[1:]:
        p = p * t + np.float32(c)
    return p


def _tc_body(slots_ref, bidx_ref):
    S, F = slots_ref.shape

    row = lax.broadcasted_iota(jnp.uint32, (S, F), 0)
    col = lax.broadcasted_iota(jnp.uint32, (S, F), 1)
    flat = row * np.uint32(F) + col
    h0, h1 = _tf_rounds(jnp.zeros((S, F), jnp.uint32), flat)
    bits = h0 ^ h1
    fb = lax.shift_right_logical(bits, np.uint32(9)) | np.uint32(0x3F800000)
    f = lax.bitcast_convert_type(fb, jnp.float32) - np.float32(1.0)
    u = jnp.maximum(_LO, f * (np.float32(1.0) - _LO) + _LO)
    w = -jnp.log1p(-u * u)
    p_small = _horner(_ERFINV_SMALL, w - np.float32(2.5))
    p_big = _horner(_ERFINV_BIG, jnp.sqrt(w) - np.float32(3.0))
    p = jnp.where(w < np.float32(5.0), p_small, p_big)
    slots_ref[:, :] = _SQRT2 * p * u

    bidx_ref[:, :] = lax.broadcasted_iota(
        jnp.int32, bidx_ref.shape, 1) // np.int32(NUM_SLOTS_PER_BATCH)


def _sc_seg_body(mask_hbm, seg_hbm, mask_v, seg_v, sem):
    cid = lax.axis_index("c")
    sid = lax.axis_index("s")
    w = sid * 2 + cid
    b = w // 4
    HW = mask_v.shape[0]

    pltpu.sync_copy(mask_hbm.at[b], mask_v)
    for i in range(HW // 16):
        v = mask_v[pl.ds(i * 16, 16)] * np.float32(_FILL)
        seg_v[0, pl.ds(i * 16, 16)] = v
        seg_v[1, pl.ds(i * 16, 16)] = v
        seg_v[2, pl.ds(i * 16, 16)] = v
    copies = [
        pltpu.async_copy(seg_v, seg_hbm.at[w * 16 + k], sem)
        for k in range(16)
    ]
    for cp in copies:
        cp.wait()


def kernel(features, feature_masks):
    B, H, W = feature_masks.shape
    S = NUM_SLOTS_PER_BATCH * B
    HW = H * W
    mask = feature_masks.reshape(B, HW).astype(jnp.float32)

    slots, bidx = pl.pallas_call(
        _tc_body,
        out_shape=[
            jax.ShapeDtypeStruct((S, _FEAT), jnp.float32),
            jax.ShapeDtypeStruct((1, S), jnp.int32),
        ],
    )()

    mesh = plsc.VectorSubcoreMesh(core_axis_name="c", subcore_axis_name="s",
                                  num_cores=2, num_subcores=16)
    seg = pl.kernel(
        _sc_seg_body,
        out_type=jax.ShapeDtypeStruct((S, 3, HW), jnp.float32),
        mesh=mesh,
        scratch_types=[
            pltpu.VMEM((HW,), jnp.float32),
            pltpu.VMEM((3, HW), jnp.float32),
            pltpu.SemaphoreType.DMA,
        ],
    )(mask)

    return slots, bidx.reshape(S), seg


# speedup vs baseline: 4.5773x; 4.5773x over previous
import numpy as np
import jax
import jax.numpy as jnp
from jax import lax
from jax.experimental import pallas as pl

NUM_SLOTS_PER_BATCH = 64
_FEAT = 256
_FILL = -1000000.0

_ROT_A = (13, 15, 26, 6)
_ROT_B = (17, 29, 16, 24)


def _np_threefry2x32(k0, k1, x0, x1):
    k0 = np.uint32(k0); k1 = np.uint32(k1)
    k2 = np.uint32(k0 ^ k1 ^ np.uint32(0x1BD11BDA))
    ks = [k0, k1, k2]
    rots = [_ROT_A, _ROT_B]
    with np.errstate(over="ignore"):
        x0 = np.uint32(x0 + k0); x1 = np.uint32(x1 + k1)
        for i in range(5):
            for r in rots[i % 2]:
                x0 = np.uint32(x0 + x1)
                x1 = np.uint32((x1 << np.uint32(r)) | (x1 >> np.uint32(32 - r)))
                x1 = np.uint32(x0 ^ x1)
            x0 = np.uint32(x0 + ks[(i + 1) % 3])
            x1 = np.uint32(x1 + ks[(i + 2) % 3] + np.uint32(i + 1))
    return x0, x1


_KS0_NP, _KS1_NP = _np_threefry2x32(0, 42, np.uint32(0), np.uint32(2))
_KS0 = np.uint32(_KS0_NP)
_KS1 = np.uint32(_KS1_NP)
_KS2 = np.uint32(_KS0 ^ _KS1 ^ np.uint32(0x1BD11BDA))

_ERFINV_SMALL = (2.81022636e-08, 3.43273939e-07, -3.5233877e-06,
                 -4.39150654e-06, 0.00021858087, -0.00125372503,
                 -0.00417768164, 0.246640727, 1.50140941)
_ERFINV_BIG = (-0.000200214257, 0.000100950558, 0.00134934322,
               -0.00367342844, 0.00573950773, -0.0076224613,
               0.00943887047, 1.00167406, 2.83297682)

_LO = np.float32(np.nextafter(np.float32(-1.0), np.float32(0.0)))
_SQRT2 = np.float32(np.sqrt(2.0))


def _tf_rounds(x0, x1):
    ks = (_KS0, _KS1, _KS2)
    rots = (_ROT_A, _ROT_B)
    x0 = x0 + _KS0
    x1 = x1 + _KS1
    for i in range(5):
        for r in rots[i % 2]:
            x0 = x0 + x1
            x1 = lax.shift_left(x1, np.uint32(r)) | lax.shift_right_logical(
                x1, np.uint32(32 - r))
            x1 = x0 ^ x1
        x0 = x0 + ks[(i + 1) % 3]
        x1 = x1 + ks[(i + 2) % 3] + np.uint32(i + 1)
    return x0, x1


def _horner(coeffs, t):
    p = jnp.full(t.shape, np.float32(coeffs[0]), dtype=jnp.float32)
    for c in coeffs[1:]:
        p = p * t + np.float32(c)
    return p


def _body(mask_ref, slots_ref, bidx_ref, seg_ref):
    S, F = slots_ref.shape
    B, _, HW = mask_ref.shape
    NB = NUM_SLOTS_PER_BATCH

    row = lax.broadcasted_iota(jnp.uint32, (S, F), 0)
    col = lax.broadcasted_iota(jnp.uint32, (S, F), 1)
    flat = row * np.uint32(F) + col
    h0, h1 = _tf_rounds(jnp.zeros((S, F), jnp.uint32), flat)
    bits = h0 ^ h1
    fb = lax.shift_right_logical(bits, np.uint32(9)) | np.uint32(0x3F800000)
    f = lax.bitcast_convert_type(fb, jnp.float32) - np.float32(1.0)
    u = jnp.maximum(_LO, f * (np.float32(1.0) - _LO) + _LO)
    w = -jnp.log1p(-u * u)
    p_small = _horner(_ERFINV_SMALL, w - np.float32(2.5))
    p_big = _horner(_ERFINV_BIG, jnp.sqrt(w) - np.float32(3.0))
    p = jnp.where(w < np.float32(5.0), p_small, p_big)
    slots_ref[:, :] = _SQRT2 * p * u

    bidx_ref[:, :] = lax.broadcasted_iota(
        jnp.int32, bidx_ref.shape, 1) // np.int32(NB)

    for b in range(B):
        mrow = mask_ref[b, 0, :].reshape(1, 1, HW)
        seg_ref[:, b * NB:(b + 1) * NB, :] = jnp.broadcast_to(
            mrow * np.float32(_FILL), (3, NB, HW))


def kernel(features, feature_masks):
    B, H, W = feature_masks.shape
    S = NUM_SLOTS_PER_BATCH * B
    HW = H * W
    mask = feature_masks.reshape(B, 1, HW).astype(jnp.float32)
    slots, bidx, seg = pl.pallas_call(
        _body,
        out_shape=[
            jax.ShapeDtypeStruct((S, _FEAT), jnp.float32),
            jax.ShapeDtypeStruct((1, S), jnp.int32),
            jax.ShapeDtypeStruct((3, S, HW), jnp.float32),
        ],
    )(mask)
    return slots, bidx.reshape(S), seg.transpose(1, 0, 2)
